# baseline (speedup 1.0000x reference)
import functools

import jax
import jax.numpy as jnp
from jax import lax
from jax.experimental import pallas as pl
from jax.experimental.pallas import tpu as pltpu
from jax.experimental.pallas import tpu_sc as plsc

VOCAB = 1000000
EMB = 64
HID = 64
B = 1024
T = 200

_NC = 2
_NS = 16
_NW = _NC * _NS
_N = B * T
_PER_W = _N // _NW
_CH = 128
_NCH = _PER_W // _CH


def _sc_gather_body(table_hbm, idx_hbm, out_hbm, idx_v, rows_v, sem):
    wid = lax.axis_index("s") * _NC + lax.axis_index("c")
    pltpu.sync_copy(idx_hbm.at[wid], idx_v)
    base = wid * _PER_W

    def chunk(j, carry):
        pltpu.async_copy(table_hbm.at[idx_v.at[j]], rows_v, sem).wait()
        pltpu.sync_copy(rows_v, out_hbm.at[pl.ds(base + j * _CH, _CH)])
        return carry

    lax.fori_loop(0, _NCH, chunk, 0)


@functools.lru_cache(maxsize=1)
def _sc_gather():
    return pl.kernel(
        _sc_gather_body,
        out_type=jax.ShapeDtypeStruct((_N, EMB), jnp.bfloat16),
        mesh=plsc.VectorSubcoreMesh(core_axis_name="c", subcore_axis_name="s"),
        scratch_types=[
            pltpu.VMEM((_NCH, _CH), jnp.int32),
            pltpu.VMEM((_CH, EMB), jnp.bfloat16),
            pltpu.SemaphoreType.DMA,
        ],
        compiler_params=pltpu.CompilerParams(use_tc_tiling_on_sc=False),
    )


def _lstm_body(emb_ref, idx_ref, Wc_ref, b_ref, W1_ref, b1_ref, W2_ref,
               b2_ref, out_ref, h_ref, c_ref):
    t = pl.program_id(0)

    @pl.when(t == 0)
    def _init():
        h_ref[...] = jnp.zeros_like(h_ref)
        c_ref[...] = jnp.zeros_like(c_ref)

    x = emb_ref[0]
    h = h_ref[...]
    c = c_ref[...]
    xh = jnp.concatenate([x, h.astype(jnp.bfloat16)], axis=1)
    z = jnp.dot(xh, Wc_ref[...],
                preferred_element_type=jnp.float32) + b_ref[...]
    i = jax.nn.sigmoid(z[:, :HID])
    f = jax.nn.sigmoid(z[:, HID:2 * HID])
    g = jnp.tanh(z[:, 2 * HID:3 * HID])
    o = jax.nn.sigmoid(z[:, 3 * HID:])
    c_new = f * c + i * g
    h_new = o * jnp.tanh(c_new)
    m = idx_ref[0] != 0
    h_ref[...] = jnp.where(m, h_new, h)
    c_ref[...] = jnp.where(m, c_new, c)

    @pl.when(t == T - 1)
    def _head():
        d = jax.nn.relu(jnp.dot(h_ref[...], W1_ref[...],
                                preferred_element_type=jnp.float32)
                        + b1_ref[...])
        out_ref[...] = jax.nn.sigmoid(
            jnp.dot(d, W2_ref[...], preferred_element_type=jnp.float32)
            + b2_ref[...])


def _lstm_call(emb, idx3, Wc, b2d, W1, b1_2d, W2, b2_2d, interpret=False):
    return pl.pallas_call(
        _lstm_body,
        grid=(T,),
        in_specs=[
            pl.BlockSpec((1, B, EMB), lambda t: (t, 0, 0)),
            pl.BlockSpec((1, B, 1), lambda t: (t, 0, 0)),
            pl.BlockSpec((EMB + HID, 4 * HID), lambda t: (0, 0)),
            pl.BlockSpec((1, 4 * HID), lambda t: (0, 0)),
            pl.BlockSpec((HID, 64), lambda t: (0, 0)),
            pl.BlockSpec((1, 64), lambda t: (0, 0)),
            pl.BlockSpec((64, 1), lambda t: (0, 0)),
            pl.BlockSpec((1, 1), lambda t: (0, 0)),
        ],
        out_specs=pl.BlockSpec((B, 1), lambda t: (0, 0)),
        out_shape=jax.ShapeDtypeStruct((B, 1), jnp.float32),
        scratch_shapes=[
            pltpu.VMEM((B, HID), jnp.float32),
            pltpu.VMEM((B, HID), jnp.float32),
        ],
        compiler_params=pltpu.CompilerParams(
            dimension_semantics=("arbitrary",)),
        interpret=interpret,
    )(emb, idx3, Wc, b2d, W1, b1_2d, W2, b2_2d)


def kernel(indices, table, W_i, W_h, b, W1, b1, W2, b2):
    table_bf = table.astype(jnp.bfloat16)
    idx_t = jnp.transpose(indices.astype(jnp.int32), (1, 0))
    flat_idx = idx_t.reshape(_NW, _NCH, _CH)
    emb_flat = _sc_gather()(table_bf, flat_idx)
    emb = emb_flat.reshape(T, B, EMB)

    idx3 = idx_t.reshape(T, B, 1)
    Wc = jnp.concatenate([W_i, W_h], axis=0).astype(jnp.bfloat16)
    out = _lstm_call(emb, idx3, Wc, b.reshape(1, -1), W1,
                     b1.reshape(1, -1), W2, b2.reshape(1, -1))
    return out


# speedup vs baseline: 1.2398x; 1.2398x over previous
import functools

import jax
import jax.numpy as jnp
from jax import lax
from jax.experimental import pallas as pl
from jax.experimental.pallas import tpu as pltpu
from jax.experimental.pallas import tpu_sc as plsc

VOCAB = 1000000
EMB = 64
HID = 64
B = 1024
T = 200

_NC = 2
_NS = 16
_NW = _NC * _NS
_N = B * T
_PER_W = _N // _NW
_CH = 128
_NCH = _PER_W // _CH


def _sc_gather_body(table_hbm, idx_hbm, out_hbm, idx_v, rows_v, sem):
    wid = lax.axis_index("s") * _NC + lax.axis_index("c")
    pltpu.sync_copy(idx_hbm.at[wid], idx_v)
    base = wid * _PER_W

    def chunk(j, carry):
        pltpu.async_copy(
            table_hbm.at[idx_v.at[pl.ds(j * _CH, _CH)]], rows_v, sem).wait()
        pltpu.sync_copy(rows_v, out_hbm.at[pl.ds(base + j * _CH, _CH)])
        return carry

    lax.fori_loop(0, _NCH, chunk, 0)


@functools.lru_cache(maxsize=1)
def _sc_gather():
    return pl.kernel(
        _sc_gather_body,
        out_type=jax.ShapeDtypeStruct((_N, 2 * EMB), jnp.float32),
        mesh=plsc.VectorSubcoreMesh(core_axis_name="c", subcore_axis_name="s"),
        scratch_types=[
            pltpu.VMEM((_PER_W,), jnp.int32),
            pltpu.VMEM((_CH, 2 * EMB), jnp.float32),
            pltpu.SemaphoreType.DMA,
        ],
        compiler_params=pltpu.CompilerParams(use_tc_tiling_on_sc=True),
    )


def _lstm_body(emb_ref, idx_ref, Wc_ref, b_ref, W1_ref, b1_ref, W2_ref,
               b2_ref, out_ref, h_ref, c_ref):
    t = pl.program_id(0)

    @pl.when(t == 0)
    def _init():
        h_ref[...] = jnp.zeros_like(h_ref)
        c_ref[...] = jnp.zeros_like(c_ref)

    x2 = emb_ref[0]
    idxv = idx_ref[0]
    odd = (idxv % 2) == 1
    x = jnp.where(odd, x2[:, EMB:], x2[:, :EMB]).astype(jnp.bfloat16)
    h = h_ref[...]
    c = c_ref[...]
    xh = jnp.concatenate([x, h.astype(jnp.bfloat16)], axis=1)
    z = jnp.dot(xh, Wc_ref[...],
                preferred_element_type=jnp.float32) + b_ref[...]
    i = jax.nn.sigmoid(z[:, :HID])
    f = jax.nn.sigmoid(z[:, HID:2 * HID])
    g = jnp.tanh(z[:, 2 * HID:3 * HID])
    o = jax.nn.sigmoid(z[:, 3 * HID:])
    c_new = f * c + i * g
    h_new = o * jnp.tanh(c_new)
    m = idxv != 0
    h_ref[...] = jnp.where(m, h_new, h)
    c_ref[...] = jnp.where(m, c_new, c)

    @pl.when(t == T - 1)
    def _head():
        d = jax.nn.relu(jnp.dot(h_ref[...], W1_ref[...],
                                preferred_element_type=jnp.float32)
                        + b1_ref[...])
        out_ref[...] = jax.nn.sigmoid(
            jnp.dot(d, W2_ref[...], preferred_element_type=jnp.float32)
            + b2_ref[...])


def _lstm_call(emb, idx3, Wc, b2d, W1, b1_2d, W2, b2_2d, interpret=False):
    return pl.pallas_call(
        _lstm_body,
        grid=(T,),
        in_specs=[
            pl.BlockSpec((1, B, 2 * EMB), lambda t: (t, 0, 0)),
            pl.BlockSpec((1, B, 1), lambda t: (t, 0, 0)),
            pl.BlockSpec((EMB + HID, 4 * HID), lambda t: (0, 0)),
            pl.BlockSpec((1, 4 * HID), lambda t: (0, 0)),
            pl.BlockSpec((HID, 64), lambda t: (0, 0)),
            pl.BlockSpec((1, 64), lambda t: (0, 0)),
            pl.BlockSpec((64, 1), lambda t: (0, 0)),
            pl.BlockSpec((1, 1), lambda t: (0, 0)),
        ],
        out_specs=pl.BlockSpec((B, 1), lambda t: (0, 0)),
        out_shape=jax.ShapeDtypeStruct((B, 1), jnp.float32),
        scratch_shapes=[
            pltpu.VMEM((B, HID), jnp.float32),
            pltpu.VMEM((B, HID), jnp.float32),
        ],
        compiler_params=pltpu.CompilerParams(
            dimension_semantics=("arbitrary",)),
        interpret=interpret,
    )(emb, idx3, Wc, b2d, W1, b1_2d, W2, b2_2d)


def kernel(indices, table, W_i, W_h, b, W1, b1, W2, b2):
    table_p = table.reshape(VOCAB // 2, 2 * EMB)
    idx_t = jnp.transpose(indices.astype(jnp.int32), (1, 0))
    pair_idx = (idx_t // 2).reshape(_NW, _PER_W)
    emb_flat = _sc_gather()(table_p, pair_idx)
    emb = emb_flat.reshape(T, B, 2 * EMB)

    idx3 = idx_t.reshape(T, B, 1)
    Wc = jnp.concatenate([W_i, W_h], axis=0).astype(jnp.bfloat16)
    out = _lstm_call(emb, idx3, Wc, b.reshape(1, -1), W1,
                     b1.reshape(1, -1), W2, b2.reshape(1, -1))
    return out


# speedup vs baseline: 1.3225x; 1.0667x over previous
import functools

import jax
import jax.numpy as jnp
from jax import lax
from jax.experimental import pallas as pl
from jax.experimental.pallas import tpu as pltpu
from jax.experimental.pallas import tpu_sc as plsc

VOCAB = 1000000
EMB = 64
HID = 64
B = 1024
T = 200

_NC = 2
_NS = 16
_NW = _NC * _NS
_N = B * T
_PER_W = _N // _NW
_CH = 128
_NCH = _PER_W // _CH


def _sc_gather_body(table_hbm, idx_hbm, out_hbm, idx_v, rows_v, sem):
    wid = lax.axis_index("s") * _NC + lax.axis_index("c")
    pltpu.sync_copy(idx_hbm.at[wid], idx_v)
    base = wid * _PER_W

    def chunk(j, carry):
        pltpu.async_copy(
            table_hbm.at[idx_v.at[pl.ds(j * _CH, _CH)]], rows_v, sem).wait()
        pltpu.sync_copy(rows_v, out_hbm.at[pl.ds(base + j * _CH, _CH)])
        return carry

    lax.fori_loop(0, _NCH, chunk, 0)


@functools.lru_cache(maxsize=1)
def _sc_gather():
    return pl.kernel(
        _sc_gather_body,
        out_type=jax.ShapeDtypeStruct((_N, EMB), jnp.float32),
        mesh=plsc.VectorSubcoreMesh(core_axis_name="c", subcore_axis_name="s"),
        scratch_types=[
            pltpu.VMEM((_PER_W,), jnp.int32),
            pltpu.VMEM((_CH, EMB), jnp.float32),
            pltpu.SemaphoreType.DMA,
        ],
        compiler_params=pltpu.CompilerParams(use_tc_tiling_on_sc=False),
    )


def _lstm_body(emb_ref, idx_ref, Wc_ref, b_ref, W1_ref, b1_ref, W2_ref,
               b2_ref, out_ref, h_ref, c_ref):
    t = pl.program_id(0)

    @pl.when(t == 0)
    def _init():
        h_ref[...] = jnp.zeros_like(h_ref)
        c_ref[...] = jnp.zeros_like(c_ref)

    x2 = emb_ref[0]
    idxv = idx_ref[0]
    x = jnp.concatenate([x2[:, :EMB], x2[:, EMB:]],
                        axis=0).astype(jnp.bfloat16)
    h = h_ref[...]
    c = c_ref[...]
    xh = jnp.concatenate([x, h.astype(jnp.bfloat16)], axis=1)
    z = jnp.dot(xh, Wc_ref[...],
                preferred_element_type=jnp.float32) + b_ref[...]
    i = jax.nn.sigmoid(z[:, :HID])
    f = jax.nn.sigmoid(z[:, HID:2 * HID])
    g = jnp.tanh(z[:, 2 * HID:3 * HID])
    o = jax.nn.sigmoid(z[:, 3 * HID:])
    c_new = f * c + i * g
    h_new = o * jnp.tanh(c_new)
    m = idxv != 0
    h_ref[...] = jnp.where(m, h_new, h)
    c_ref[...] = jnp.where(m, c_new, c)

    @pl.when(t == T - 1)
    def _head():
        d = jax.nn.relu(jnp.dot(h_ref[...], W1_ref[...],
                                preferred_element_type=jnp.float32)
                        + b1_ref[...])
        out_ref[...] = jax.nn.sigmoid(
            jnp.dot(d, W2_ref[...], preferred_element_type=jnp.float32)
            + b2_ref[...])


def _lstm_call(emb, idx3, Wc, b2d, W1, b1_2d, W2, b2_2d, interpret=False):
    return pl.pallas_call(
        _lstm_body,
        grid=(T,),
        in_specs=[
            pl.BlockSpec((1, B // 2, 2 * EMB), lambda t: (t, 0, 0)),
            pl.BlockSpec((1, B, 1), lambda t: (t, 0, 0)),
            pl.BlockSpec((EMB + HID, 4 * HID), lambda t: (0, 0)),
            pl.BlockSpec((1, 4 * HID), lambda t: (0, 0)),
            pl.BlockSpec((HID, 64), lambda t: (0, 0)),
            pl.BlockSpec((1, 64), lambda t: (0, 0)),
            pl.BlockSpec((64, 1), lambda t: (0, 0)),
            pl.BlockSpec((1, 1), lambda t: (0, 0)),
        ],
        out_specs=pl.BlockSpec((B, 1), lambda t: (0, 0)),
        out_shape=jax.ShapeDtypeStruct((B, 1), jnp.float32),
        scratch_shapes=[
            pltpu.VMEM((B, HID), jnp.float32),
            pltpu.VMEM((B, HID), jnp.float32),
        ],
        compiler_params=pltpu.CompilerParams(
            dimension_semantics=("arbitrary",)),
        interpret=interpret,
    )(emb, idx3, Wc, b2d, W1, b1_2d, W2, b2_2d)


def kernel(indices, table, W_i, W_h, b, W1, b1, W2, b2):
    idx_t = jnp.transpose(indices.astype(jnp.int32), (1, 0))
    idx_il = jnp.stack([idx_t[:, :B // 2], idx_t[:, B // 2:]], axis=2)
    flat_idx = idx_il.reshape(_NW, _PER_W)
    emb_flat = _sc_gather()(table, flat_idx)
    emb = emb_flat.reshape(T, B // 2, 2 * EMB)

    idx3 = idx_t.reshape(T, B, 1)
    Wc = jnp.concatenate([W_i, W_h], axis=0).astype(jnp.bfloat16)
    out = _lstm_call(emb, idx3, Wc, b.reshape(1, -1), W1,
                     b1.reshape(1, -1), W2, b2.reshape(1, -1))
    return out


# speedup vs baseline: 1.7617x; 1.3321x over previous
import functools

import jax
import jax.numpy as jnp
from jax import lax
from jax.experimental import pallas as pl
from jax.experimental.pallas import tpu as pltpu
from jax.experimental.pallas import tpu_sc as plsc

VOCAB = 1000000
EMB = 64
HID = 64
B = 1024
T = 200

_NC = 2
_NS = 16
_NW = _NC * _NS
_N = B * T
_PER_W = _N // _NW
_CH = 800
_NCH = _PER_W // _CH


def _sc_gather_body(table_hbm, idx_hbm, out_hbm, idx_v, rows_v, sem):
    wid = lax.axis_index("s") * _NC + lax.axis_index("c")
    base = wid * _PER_W
    pltpu.sync_copy(idx_hbm.at[pl.ds(base, _PER_W)], idx_v)

    def chunk(c, carry):
        cb = base + c * _CH

        def grp(g, carry2):
            v = idx_v[pl.ds(c * _CH + g * 16, 16)]
            for k in range(16):
                tok = v[k]
                pltpu.async_copy(table_hbm.at[pl.ds(tok, 1)],
                                 rows_v.at[pl.ds(g * 16 + k, 1)], sem)
            return carry2

        lax.fori_loop(0, _CH // 16, grp, 0)
        pltpu.make_async_copy(table_hbm.at[pl.ds(0, _CH)], rows_v, sem).wait()
        pltpu.sync_copy(rows_v, out_hbm.at[pl.ds(cb, _CH)])
        return carry

    lax.fori_loop(0, _NCH, chunk, 0)


@functools.lru_cache(maxsize=1)
def _sc_gather():
    return pl.kernel(
        _sc_gather_body,
        out_type=jax.ShapeDtypeStruct((_N, EMB), jnp.float32),
        mesh=plsc.VectorSubcoreMesh(core_axis_name="c", subcore_axis_name="s"),
        scratch_types=[
            pltpu.VMEM((_PER_W,), jnp.int32),
            pltpu.VMEM((_CH, EMB), jnp.float32),
            pltpu.SemaphoreType.DMA,
        ],
        compiler_params=pltpu.CompilerParams(use_tc_tiling_on_sc=True),
    )


def _lstm_body(emb_ref, idx_ref, Wc_ref, b_ref, W1_ref, b1_ref, W2_ref,
               b2_ref, out_ref, h_ref, c_ref):
    t = pl.program_id(0)

    @pl.when(t == 0)
    def _init():
        h_ref[...] = jnp.zeros_like(h_ref)
        c_ref[...] = jnp.zeros_like(c_ref)

    x = emb_ref[0].astype(jnp.bfloat16)
    idxv = idx_ref[0]
    h = h_ref[...]
    c = c_ref[...]
    xh = jnp.concatenate([x, h.astype(jnp.bfloat16)], axis=1)
    z = jnp.dot(xh, Wc_ref[...],
                preferred_element_type=jnp.float32) + b_ref[...]
    i = jax.nn.sigmoid(z[:, :HID])
    f = jax.nn.sigmoid(z[:, HID:2 * HID])
    g = jnp.tanh(z[:, 2 * HID:3 * HID])
    o = jax.nn.sigmoid(z[:, 3 * HID:])
    c_new = f * c + i * g
    h_new = o * jnp.tanh(c_new)
    m = idxv != 0
    h_ref[...] = jnp.where(m, h_new, h)
    c_ref[...] = jnp.where(m, c_new, c)

    @pl.when(t == T - 1)
    def _head():
        d = jax.nn.relu(jnp.dot(h_ref[...], W1_ref[...],
                                preferred_element_type=jnp.float32)
                        + b1_ref[...])
        out_ref[...] = jax.nn.sigmoid(
            jnp.dot(d, W2_ref[...], preferred_element_type=jnp.float32)
            + b2_ref[...])


def _lstm_call(emb, idx3, Wc, b2d, W1, b1_2d, W2, b2_2d, interpret=False):
    return pl.pallas_call(
        _lstm_body,
        grid=(T,),
        in_specs=[
            pl.BlockSpec((1, B, EMB), lambda t: (t, 0, 0)),
            pl.BlockSpec((1, B, 1), lambda t: (t, 0, 0)),
            pl.BlockSpec((EMB + HID, 4 * HID), lambda t: (0, 0)),
            pl.BlockSpec((1, 4 * HID), lambda t: (0, 0)),
            pl.BlockSpec((HID, 64), lambda t: (0, 0)),
            pl.BlockSpec((1, 64), lambda t: (0, 0)),
            pl.BlockSpec((64, 1), lambda t: (0, 0)),
            pl.BlockSpec((1, 1), lambda t: (0, 0)),
        ],
        out_specs=pl.BlockSpec((B, 1), lambda t: (0, 0)),
        out_shape=jax.ShapeDtypeStruct((B, 1), jnp.float32),
        scratch_shapes=[
            pltpu.VMEM((B, HID), jnp.float32),
            pltpu.VMEM((B, HID), jnp.float32),
        ],
        compiler_params=pltpu.CompilerParams(
            dimension_semantics=("arbitrary",)),
        interpret=interpret,
    )(emb, idx3, Wc, b2d, W1, b1_2d, W2, b2_2d)


def kernel(indices, table, W_i, W_h, b, W1, b1, W2, b2):
    idx_t = jnp.transpose(indices.astype(jnp.int32), (1, 0))
    flat_idx = idx_t.reshape(_N)
    emb_flat = _sc_gather()(table, flat_idx)
    emb = emb_flat.reshape(T, B, EMB)

    idx3 = idx_t.reshape(T, B, 1)
    Wc = jnp.concatenate([W_i, W_h], axis=0).astype(jnp.bfloat16)
    out = _lstm_call(emb, idx3, Wc, b.reshape(1, -1), W1,
                     b1.reshape(1, -1), W2, b2.reshape(1, -1))
    return out


# speedup vs baseline: 1.7848x; 1.0131x over previous
import functools

import jax
import jax.numpy as jnp
from jax import lax
from jax.experimental import pallas as pl
from jax.experimental.pallas import tpu as pltpu
from jax.experimental.pallas import tpu_sc as plsc

VOCAB = 1000000
EMB = 64
HID = 64
B = 1024
T = 200

_NC = 2
_NS = 16
_NW = _NC * _NS
_N = B * T
_CH = 800


def _sc_gather_body(n_rows, table_hbm, idx_hbm, out_hbm, idx_v, rows_v, sem):
    per_w = n_rows // _NW
    n_ch = per_w // _CH
    wid = lax.axis_index("s") * _NC + lax.axis_index("c")
    base = wid * per_w
    pltpu.sync_copy(idx_hbm.at[pl.ds(base, per_w)], idx_v)

    def chunk(c, carry):
        cb = base + c * _CH

        def grp(g, carry2):
            v = idx_v[pl.ds(c * _CH + g * 16, 16)]
            for k in range(16):
                tok = v[k]
                pltpu.async_copy(table_hbm.at[pl.ds(tok, 1)],
                                 rows_v.at[pl.ds(g * 16 + k, 1)], sem)
            return carry2

        lax.fori_loop(0, _CH // 16, grp, 0)
        pltpu.make_async_copy(table_hbm.at[pl.ds(0, _CH)], rows_v, sem).wait()
        pltpu.sync_copy(rows_v, out_hbm.at[pl.ds(cb, _CH)])
        return carry

    lax.fori_loop(0, n_ch, chunk, 0)


@functools.lru_cache(maxsize=4)
def _sc_gather(n_rows):
    return pl.kernel(
        functools.partial(_sc_gather_body, n_rows),
        out_type=jax.ShapeDtypeStruct((n_rows, EMB), jnp.float32),
        mesh=plsc.VectorSubcoreMesh(core_axis_name="c", subcore_axis_name="s"),
        scratch_types=[
            pltpu.VMEM((n_rows // _NW,), jnp.int32),
            pltpu.VMEM((_CH, EMB), jnp.float32),
            pltpu.SemaphoreType.DMA,
        ],
        compiler_params=pltpu.CompilerParams(use_tc_tiling_on_sc=True),
    )


def _lstm_body(t_chunk, emb_ref, idx_ref, h0_ref, c0_ref, Wc_ref, b_ref,
               W1_ref, b1_ref, W2_ref, b2_ref, out_ref, ho_ref, co_ref,
               h_ref, c_ref):
    t = pl.program_id(0)

    @pl.when(t == 0)
    def _init():
        h_ref[...] = h0_ref[...]
        c_ref[...] = c0_ref[...]

    x = emb_ref[0].astype(jnp.bfloat16)
    idxv = idx_ref[0]
    h = h_ref[...]
    c = c_ref[...]
    xh = jnp.concatenate([x, h.astype(jnp.bfloat16)], axis=1)
    z = jnp.dot(xh, Wc_ref[...],
                preferred_element_type=jnp.float32) + b_ref[...]
    i = jax.nn.sigmoid(z[:, :HID])
    f = jax.nn.sigmoid(z[:, HID:2 * HID])
    g = jnp.tanh(z[:, 2 * HID:3 * HID])
    o = jax.nn.sigmoid(z[:, 3 * HID:])
    c_new = f * c + i * g
    h_new = o * jnp.tanh(c_new)
    m = idxv != 0
    h_ref[...] = jnp.where(m, h_new, h)
    c_ref[...] = jnp.where(m, c_new, c)

    @pl.when(t == t_chunk - 1)
    def _head():
        ho_ref[...] = h_ref[...]
        co_ref[...] = c_ref[...]
        d = jax.nn.relu(jnp.dot(h_ref[...], W1_ref[...],
                                preferred_element_type=jnp.float32)
                        + b1_ref[...])
        out_ref[...] = jax.nn.sigmoid(
            jnp.dot(d, W2_ref[...], preferred_element_type=jnp.float32)
            + b2_ref[...])


def _lstm_call(emb, idx3, h0, c0, Wc, b2d, W1, b1_2d, W2, b2_2d,
               interpret=False):
    t_chunk = emb.shape[0]
    full = lambda t: (0, 0)
    return pl.pallas_call(
        functools.partial(_lstm_body, t_chunk),
        grid=(t_chunk,),
        in_specs=[
            pl.BlockSpec((1, B, EMB), lambda t: (t, 0, 0)),
            pl.BlockSpec((1, B, 1), lambda t: (t, 0, 0)),
            pl.BlockSpec((B, HID), full),
            pl.BlockSpec((B, HID), full),
            pl.BlockSpec((EMB + HID, 4 * HID), full),
            pl.BlockSpec((1, 4 * HID), full),
            pl.BlockSpec((HID, 64), full),
            pl.BlockSpec((1, 64), full),
            pl.BlockSpec((64, 1), full),
            pl.BlockSpec((1, 1), full),
        ],
        out_specs=[
            pl.BlockSpec((B, 1), full),
            pl.BlockSpec((B, HID), full),
            pl.BlockSpec((B, HID), full),
        ],
        out_shape=[
            jax.ShapeDtypeStruct((B, 1), jnp.float32),
            jax.ShapeDtypeStruct((B, HID), jnp.float32),
            jax.ShapeDtypeStruct((B, HID), jnp.float32),
        ],
        scratch_shapes=[
            pltpu.VMEM((B, HID), jnp.float32),
            pltpu.VMEM((B, HID), jnp.float32),
        ],
        compiler_params=pltpu.CompilerParams(
            dimension_semantics=("arbitrary",)),
        interpret=interpret,
    )(emb, idx3, h0, c0, Wc, b2d, W1, b1_2d, W2, b2_2d)


_NSPLIT = 2
_TC = T // _NSPLIT


def kernel(indices, table, W_i, W_h, b, W1, b1, W2, b2):
    idx_t = jnp.transpose(indices.astype(jnp.int32), (1, 0))
    idx3 = idx_t.reshape(T, B, 1)
    Wc = jnp.concatenate([W_i, W_h], axis=0).astype(jnp.bfloat16)
    b2d, b1_2d, b2_2d = b.reshape(1, -1), b1.reshape(1, -1), b2.reshape(1, -1)

    h = jnp.zeros((B, HID), jnp.float32)
    c = jnp.zeros((B, HID), jnp.float32)
    out = None
    embs = []
    for k in range(_NSPLIT):
        flat_idx = idx_t[k * _TC:(k + 1) * _TC].reshape(_TC * B)
        emb_flat = _sc_gather(_TC * B)(table, flat_idx)
        embs.append(emb_flat.reshape(_TC, B, EMB))
    for k in range(_NSPLIT):
        out, h, c = _lstm_call(embs[k], idx3[k * _TC:(k + 1) * _TC], h, c,
                               Wc, b2d, W1, b1_2d, W2, b2_2d)
    return out
